# speedup vs baseline: 1.6093x; 1.6093x over previous
import functools

import jax
import jax.numpy as jnp
from jax import lax
from jax.experimental import pallas as pl
from jax.experimental.pallas import tpu as pltpu
from jax.experimental.pallas import tpu_sc as plsc

B, S, H = 32, 512, 768
N = B * S
VOCAB = 100000
OFFSET = 2
EPS = 1e-5

NW = 32
TOK_PER_W = N // NW
CHUNK = 64
NCHUNK = TOK_PER_W // CHUNK


def _sc_gather_body(ids_hbm, table_hbm, out_hbm, idx_v, rows_v,
                    sem_g0, sem_g1, sem_s0, sem_s1):
    wid = lax.axis_index("s") * 2 + lax.axis_index("c")
    base = wid * TOK_PER_W
    pltpu.sync_copy(ids_hbm.at[wid], idx_v)

    g_sems = (sem_g0, sem_g1)
    s_sems = (sem_s0, sem_s1)

    def gather_start(i, slot):
        return pltpu.async_copy(
            table_hbm.at[idx_v.at[i]], rows_v.at[slot], g_sems[slot])

    def scatter_start(i, slot):
        return pltpu.async_copy(
            rows_v.at[slot], out_hbm.at[pl.ds(base + i * CHUNK, CHUNK)],
            s_sems[slot])

    gathers = [None, None]
    scatters = [None, None]
    gathers[0] = gather_start(0, 0)
    for i in range(NCHUNK):
        slot = i % 2
        nxt = (i + 1) % 2
        gathers[slot].wait()
        if i + 1 < NCHUNK:
            if scatters[nxt] is not None:
                scatters[nxt].wait()
            gathers[nxt] = gather_start(i + 1, nxt)
        scatters[slot] = scatter_start(i, slot)
    scatters[0].wait()
    scatters[1].wait()


_sc_gather = functools.partial(
    pl.kernel,
    mesh=plsc.VectorSubcoreMesh(core_axis_name="c", subcore_axis_name="s"),
    out_type=jax.ShapeDtypeStruct((N, H), jnp.float32),
    scratch_types=[
        pltpu.VMEM((NCHUNK, CHUNK), jnp.int32),
        pltpu.VMEM((2, CHUNK, H), jnp.float32),
        pltpu.SemaphoreType.DMA,
        pltpu.SemaphoreType.DMA,
        pltpu.SemaphoreType.DMA,
        pltpu.SemaphoreType.DMA,
    ],
)(_sc_gather_body)


def _ln_body(x_ref, pos_ref, g_ref, b_ref, o_ref):
    x = x_ref[...] + pos_ref[...]
    mean = jnp.mean(x, axis=-1, keepdims=True)
    xc = x - mean
    var = jnp.mean(xc * xc, axis=-1, keepdims=True)
    o_ref[...] = xc * lax.rsqrt(var + EPS) * g_ref[...] + b_ref[...]


def _ln(gathered, pos, gamma2d, beta2d):
    return pl.pallas_call(
        _ln_body,
        grid=(N // S,),
        in_specs=[
            pl.BlockSpec((S, H), lambda i: (i, 0)),
            pl.BlockSpec((S, H), lambda i: (0, 0)),
            pl.BlockSpec((1, H), lambda i: (0, 0)),
            pl.BlockSpec((1, H), lambda i: (0, 0)),
        ],
        out_specs=pl.BlockSpec((S, H), lambda i: (i, 0)),
        out_shape=jax.ShapeDtypeStruct((N, H), jnp.float32),
    )(gathered, pos, gamma2d, beta2d)


def kernel(token_ids, word_embeddings, position_embeddings, ln_gamma, ln_beta):
    ids = token_ids.reshape(NW, NCHUNK, CHUNK).astype(jnp.int32)
    gathered = _sc_gather(ids, word_embeddings)
    pos = lax.slice_in_dim(position_embeddings, OFFSET, OFFSET + S, axis=0)
    out = _ln(gathered, pos,
              ln_gamma.reshape(1, H), ln_beta.reshape(1, H))
    return out.reshape(B, S, H)


# speedup vs baseline: 1.6422x; 1.0205x over previous
import functools

import jax
import jax.numpy as jnp
from jax import lax
from jax.experimental import pallas as pl
from jax.experimental.pallas import tpu as pltpu
from jax.experimental.pallas import tpu_sc as plsc

B, S, H = 32, 512, 768
N = B * S
OFFSET = 2
EPS = 1e-5

NW = 32
CHUNK = 64
P = 4
ROWS_P = N // P
TOK_PER_W = ROWS_P // NW
NCHUNK = TOK_PER_W // CHUNK


def _sc_gather_body(ids_hbm, table_hbm, out_hbm, idx_v, rows_v,
                    sem_g0, sem_g1, sem_s0, sem_s1):
    wid = lax.axis_index("s") * 2 + lax.axis_index("c")
    base = wid * TOK_PER_W
    pltpu.sync_copy(ids_hbm.at[wid], idx_v)

    g_sems = (sem_g0, sem_g1)
    s_sems = (sem_s0, sem_s1)

    def gather_start(i, slot):
        return pltpu.async_copy(
            table_hbm.at[idx_v.at[i]], rows_v.at[slot], g_sems[slot])

    def scatter_start(i, slot):
        return pltpu.async_copy(
            rows_v.at[slot], out_hbm.at[pl.ds(base + i * CHUNK, CHUNK)],
            s_sems[slot])

    gathers = [None, None]
    scatters = [None, None]
    gathers[0] = gather_start(0, 0)
    for i in range(NCHUNK):
        slot = i % 2
        nxt = (i + 1) % 2
        gathers[slot].wait()
        if i + 1 < NCHUNK:
            if scatters[nxt] is not None:
                scatters[nxt].wait()
            gathers[nxt] = gather_start(i + 1, nxt)
        scatters[slot] = scatter_start(i, slot)
    for s in scatters:
        if s is not None:
            s.wait()


_sc_gather = functools.partial(
    pl.kernel,
    mesh=plsc.VectorSubcoreMesh(core_axis_name="c", subcore_axis_name="s"),
    out_type=jax.ShapeDtypeStruct((ROWS_P, H), jnp.float32),
    scratch_types=[
        pltpu.VMEM((NCHUNK, CHUNK), jnp.int32),
        pltpu.VMEM((2, CHUNK, H), jnp.float32),
        pltpu.SemaphoreType.DMA,
        pltpu.SemaphoreType.DMA,
        pltpu.SemaphoreType.DMA,
        pltpu.SemaphoreType.DMA,
    ],
)(_sc_gather_body)


def _ln_body(x_ref, pos_ref, g_ref, b_ref, o_ref):
    x = x_ref[...] + pos_ref[...]
    mean = jnp.mean(x, axis=-1, keepdims=True)
    xc = x - mean
    var = jnp.mean(xc * xc, axis=-1, keepdims=True)
    o_ref[...] = xc * lax.rsqrt(var + EPS) * g_ref[...] + b_ref[...]


def _ln_body_aliased(x_ref, pos_ref, g_ref, b_ref, prev_ref, o_ref):
    _ln_body(x_ref, pos_ref, g_ref, b_ref, o_ref)


def _ln_part(p, gathered, pos, gamma2d, beta2d, prev_out):
    nblk = ROWS_P // S
    in_specs = [
        pl.BlockSpec((S, H), lambda i: (i, 0)),
        pl.BlockSpec((S, H), lambda i: (0, 0)),
        pl.BlockSpec((1, H), lambda i: (0, 0)),
        pl.BlockSpec((1, H), lambda i: (0, 0)),
    ]
    out_spec = pl.BlockSpec((S, H), lambda i, p=p: (p * nblk + i, 0))
    if p == 0:
        return pl.pallas_call(
            _ln_body,
            grid=(nblk,),
            in_specs=in_specs,
            out_specs=out_spec,
            out_shape=jax.ShapeDtypeStruct((N, H), jnp.float32),
        )(gathered, pos, gamma2d, beta2d)
    in_specs.append(pl.BlockSpec(memory_space=pl.ANY))
    return pl.pallas_call(
        _ln_body_aliased,
        grid=(nblk,),
        in_specs=in_specs,
        out_specs=out_spec,
        out_shape=jax.ShapeDtypeStruct((N, H), jnp.float32),
        input_output_aliases={4: 0},
    )(gathered, pos, gamma2d, beta2d, prev_out)


def kernel(token_ids, word_embeddings, position_embeddings, ln_gamma, ln_beta):
    ids = token_ids.reshape(P, NW, NCHUNK, CHUNK).astype(jnp.int32)
    pos = lax.slice_in_dim(position_embeddings, OFFSET, OFFSET + S, axis=0)
    gamma2d = ln_gamma.reshape(1, H)
    beta2d = ln_beta.reshape(1, H)

    gathered = [_sc_gather(ids[p], word_embeddings) for p in range(P)]
    out = None
    for p in range(P):
        out = _ln_part(p, gathered[p], pos, gamma2d, beta2d, out)
    return out.reshape(B, S, H)
